# baseline (speedup 1.0000x reference)
import jax
import jax.numpy as jnp
from jax import lax
from jax.experimental import pallas as pl
from jax.experimental.pallas import tpu as pltpu
from jax.experimental.pallas import tpu_sc as plsc

NS = 16
LB = 128
KB = 8
F32 = jnp.float32


def _round_up(a, b):
    return (a + b - 1) // b * b


def _make_sc_pass(nc, NP, RW, NBAT, gather, base_w):
    rows_per_tile = NP // NS
    nzero = rows_per_tile // LB

    mesh = plsc.VectorSubcoreMesh(
        core_axis_name="c", subcore_axis_name="s", num_cores=1,
        num_subcores=NS)

    out_type = [jax.ShapeDtypeStruct((NP, 16), F32) for _ in range(nc)]
    scratch = [
        pltpu.VMEM((KB, LB), jnp.int32),
        pltpu.VMEM((LB, 16), F32),
        pltpu.VMEM_SHARED((NP, 16), F32),
    ]
    if gather:
        scratch = [pltpu.VMEM((KB, LB), jnp.int32)] + scratch
        scratch.append(pltpu.VMEM((KB, LB, 16), F32))
        scratch.append(pltpu.SemaphoreType.DMA)
    else:
        scratch.append(pltpu.VMEM((LB, 16), F32))

    def body(*refs):
        if gather:
            s2, d2 = refs[0], refs[1]
            ytabs = refs[2:2 + nc]
            outs = refs[2 + nc:2 + 2 * nc]
            s_buf, d_buf, zbuf, acc, rows, sem = refs[2 + 2 * nc:]
        else:
            d2, ones_hbm = refs[0], refs[1]
            outs = refs[2:2 + nc]
            d_buf, zbuf, acc, obuf = refs[2 + nc:]

        sid = lax.axis_index("s")
        my0 = sid * rows_per_tile

        zvec = jnp.zeros((16,), F32)
        for r in range(LB):
            zbuf[r] = zvec
        if not gather:
            pltpu.sync_copy(ones_hbm, obuf)

        def zero_my_slice():
            for k in range(nzero):
                pltpu.sync_copy(zbuf, acc.at[pl.ds(my0 + k * LB, LB)])

        zero_my_slice()
        plsc.subcore_barrier()

        wrow = (base_w + sid) * RW

        for chunk in range(nc):
            def batch_body(i, carry):
                r0 = pl.multiple_of(wrow + i * KB, 8)
                pltpu.sync_copy(d2.at[pl.ds(r0, KB)], d_buf)
                if gather:
                    pltpu.sync_copy(s2.at[pl.ds(r0, KB)], s_buf)
                    cps = [
                        pltpu.async_copy(
                            ytabs[chunk].at[s_buf.at[j]], rows.at[j], sem)
                        for j in range(KB)
                    ]
                    for cp in cps:
                        cp.wait()
                    for j in range(KB):
                        pltpu.sync_copy(rows.at[j], acc.at[d_buf.at[j]],
                                        add=True)
                else:
                    for j in range(KB):
                        pltpu.sync_copy(obuf, acc.at[d_buf.at[j]], add=True)
                return carry

            lax.fori_loop(0, NBAT, batch_body, 0)
            plsc.subcore_barrier()
            pltpu.sync_copy(acc.at[pl.ds(my0, rows_per_tile)],
                            outs[chunk].at[pl.ds(my0, rows_per_tile)])
            if chunk < nc - 1:
                zero_my_slice()
            plsc.subcore_barrier()

    return pl.kernel(
        body, out_type=out_type, mesh=mesh,
        compiler_params=pltpu.CompilerParams(use_tc_tiling_on_sc=False),
        scratch_types=scratch)


_DOT = dict(preferred_element_type=F32, precision=lax.Precision.HIGHEST)


def _tc_prep(NP, NB, cnt0, cnt1, x_pad, W1p):
    grid = NP // NB

    def body(c0_ref, c1_ref, x_ref, w_ref, dinv_ref, y1_ref):
        c = c0_ref[:, 0:1] + c1_ref[:, 0:1] + 1.0
        dinv = lax.rsqrt(c)
        dinv16 = jnp.broadcast_to(dinv, (NB, 16))
        dinv_ref[...] = dinv16
        y1_ref[...] = jnp.dot(x_ref[...], w_ref[...], **_DOT) * dinv16

    return pl.pallas_call(
        body,
        grid=(grid,),
        in_specs=[
            pl.BlockSpec((NB, 16), lambda i: (i, 0)),
            pl.BlockSpec((NB, 16), lambda i: (i, 0)),
            pl.BlockSpec((NB, 8), lambda i: (i, 0)),
            pl.BlockSpec((8, 16), lambda i: (0, 0)),
        ],
        out_specs=[
            pl.BlockSpec((NB, 16), lambda i: (i, 0)),
            pl.BlockSpec((NB, 16), lambda i: (i, 0)),
        ],
        out_shape=[
            jax.ShapeDtypeStruct((NP, 16), F32),
            jax.ShapeDtypeStruct((NP, 16), F32),
        ],
    )(cnt0, cnt1, x_pad, W1p)


def _tc_mid(NP, NB, nci, nco, acc0s, acc1s, ys, dinv16, bt, W):
    grid = NP // NB

    def body(*refs):
        a0_refs = refs[:nci]
        a1_refs = refs[nci:2 * nci]
        y_refs = refs[2 * nci:3 * nci]
        dinv_ref, b_ref, w_ref = refs[3 * nci:3 * nci + 3]
        o_refs = refs[3 * nci + 3:]
        dinv = dinv_ref[...]
        hs = []
        for c in range(nci):
            a = a0_refs[c][...] + a1_refs[c][...] + y_refs[c][...]
            hs.append(jax.nn.relu(dinv * a + b_ref[0, 16 * c:16 * (c + 1)]))
        for oc in range(nco):
            acc = None
            for c in range(nci):
                blk = w_ref[16 * c:16 * (c + 1), 16 * oc:16 * (oc + 1)]
                t = jnp.dot(hs[c], blk, **_DOT)
                acc = t if acc is None else acc + t
            o_refs[oc][...] = acc * dinv

    in_specs = (
        [pl.BlockSpec((NB, 16), lambda i: (i, 0))] * (3 * nci)
        + [pl.BlockSpec((NB, 16), lambda i: (i, 0)),
           pl.BlockSpec((1, 16 * nci), lambda i: (0, 0)),
           pl.BlockSpec((16 * nci, 16 * nco), lambda i: (0, 0))]
    )
    return pl.pallas_call(
        body,
        grid=(grid,),
        in_specs=in_specs,
        out_specs=[pl.BlockSpec((NB, 16), lambda i: (i, 0))] * nco,
        out_shape=[jax.ShapeDtypeStruct((NP, 16), F32)] * nco,
    )(*acc0s, *acc1s, *ys, dinv16, bt, W)


def _tc_final(NP, NB, G, acc0s, acc1s, ys, dinv16, b3t, batch2, Wfc, bfc2):
    grid = NP // NB
    nci = 4

    def body(*refs):
        a0_refs = refs[:nci]
        a1_refs = refs[nci:2 * nci]
        y_refs = refs[2 * nci:3 * nci]
        dinv_ref, b_ref, bat_ref, wfc_ref, bfc_ref = refs[3 * nci:3 * nci + 5]
        out_ref = refs[3 * nci + 5]
        pooled = refs[3 * nci + 6:3 * nci + 6 + nci]
        cnt_ref = refs[3 * nci + 6 + nci]

        i = pl.program_id(0)

        @pl.when(i == 0)
        def _init():
            for c in range(nci):
                pooled[c][...] = jnp.zeros((G, 16), F32)
            cnt_ref[...] = jnp.zeros((G, 1), F32)

        dinv = dinv_ref[...]
        bat = bat_ref[...]
        gid = lax.broadcasted_iota(jnp.int32, (NB, G), 1)
        A = (bat == gid).astype(F32)
        dn = (((0,), (0,)), ((), ()))
        for c in range(nci):
            a = a0_refs[c][...] + a1_refs[c][...] + y_refs[c][...]
            h = jax.nn.relu(dinv * a + b_ref[0, 16 * c:16 * (c + 1)])
            pooled[c][...] += lax.dot_general(A, h, dn, **_DOT)
        cnt_ref[...] += lax.dot_general(A, jnp.ones((NB, 1), F32), dn, **_DOT)

        @pl.when(i == grid - 1)
        def _fin():
            cnt = jnp.clip(cnt_ref[...], 1.0, None)
            logits = None
            for c in range(nci):
                p = pooled[c][...] / cnt
                t = jnp.dot(p, wfc_ref[16 * c:16 * (c + 1), :], **_DOT)
                logits = t if logits is None else logits + t
            logits = logits + bfc_ref[...]
            m = jnp.max(logits, axis=1, keepdims=True)
            e = jnp.exp(logits - m)
            lse = m + jnp.log(jnp.sum(e, axis=1, keepdims=True))
            out_ref[...] = logits - lse

    in_specs = (
        [pl.BlockSpec((NB, 16), lambda i: (i, 0))] * (3 * nci)
        + [pl.BlockSpec((NB, 16), lambda i: (i, 0)),
           pl.BlockSpec((1, 64), lambda i: (0, 0)),
           pl.BlockSpec((NB, 1), lambda i: (i, 0)),
           pl.BlockSpec((64, 2), lambda i: (0, 0)),
           pl.BlockSpec((1, 2), lambda i: (0, 0))]
    )
    return pl.pallas_call(
        body,
        grid=(grid,),
        in_specs=in_specs,
        out_specs=pl.BlockSpec((G, 2), lambda i: (0, 0)),
        out_shape=jax.ShapeDtypeStruct((G, 2), F32),
        scratch_shapes=[pltpu.VMEM((G, 16), F32) for _ in range(nci)]
        + [pltpu.VMEM((G, 1), F32)],
    )(*acc0s, *acc1s, *ys, dinv16, b3t, batch2, Wfc, bfc2)


def kernel(x, edge_index, batch, W1, b1, W2, b2, W3, b3, Wfc, bfc):
    N = x.shape[0]
    E = edge_index.shape[1]
    G = 128
    NB = 2048
    NP = _round_up(N + 1, NB)
    NW = 2 * NS
    per_w = -(-E // NW)
    NBAT = -(-per_w // (KB * LB))
    RW = NBAT * KB
    EP = NW * RW * LB

    src = edge_index[0]
    dst = edge_index[1]
    pad_e = EP - E
    src2 = jnp.concatenate(
        [src, jnp.full((pad_e,), N, jnp.int32)]).reshape(EP // LB, LB)
    dst2 = jnp.concatenate(
        [dst, jnp.full((pad_e,), N, jnp.int32)]).reshape(EP // LB, LB)

    x_pad = jnp.zeros((NP, 8), F32).at[:N, :5].set(x)
    W1p = jnp.zeros((8, 16), F32).at[:5, :].set(W1)
    batch2 = jnp.concatenate(
        [batch.astype(jnp.int32),
         jnp.full((NP - N,), G, jnp.int32)]).reshape(NP, 1)
    b1t = b1.reshape(1, 16)
    b2t = b2.reshape(1, 32)
    b3t = b3.reshape(1, 64)
    bfc2 = bfc.reshape(1, 2)

    halves = []
    for base_w in (0, NS):
        halves.append(dict(
            deg=_make_sc_pass(1, NP, RW, NBAT, False, base_w),
            l1=_make_sc_pass(1, NP, RW, NBAT, True, base_w),
            l2=_make_sc_pass(2, NP, RW, NBAT, True, base_w),
            l3=_make_sc_pass(4, NP, RW, NBAT, True, base_w),
        ))

    ones_rows = jnp.zeros((LB, 16), F32).at[:, 0].set(1.0)
    (cnt0,) = halves[0]["deg"](dst2, ones_rows)
    (cnt1,) = halves[1]["deg"](dst2, ones_rows)
    dinv16, y1 = _tc_prep(NP, NB, cnt0, cnt1, x_pad, W1p)
    (a1_0,) = halves[0]["l1"](src2, dst2, y1)
    (a1_1,) = halves[1]["l1"](src2, dst2, y1)
    y2 = _tc_mid(NP, NB, 1, 2, [a1_0], [a1_1], [y1], dinv16, b1t, W2)
    a2_0 = halves[0]["l2"](src2, dst2, *y2)
    a2_1 = halves[1]["l2"](src2, dst2, *y2)
    y3 = _tc_mid(NP, NB, 2, 4, a2_0, a2_1, y2, dinv16, b2t, W3)
    a3_0 = halves[0]["l3"](src2, dst2, *y3)
    a3_1 = halves[1]["l3"](src2, dst2, *y3)
    return _tc_final(NP, NB, G, a3_0, a3_1, y3, dinv16, b3t, batch2, Wfc,
                     bfc2)


# speedup vs baseline: 1.4779x; 1.4779x over previous
import jax
import jax.numpy as jnp
from jax import lax
from jax.experimental import pallas as pl
from jax.experimental.pallas import tpu as pltpu
from jax.experimental.pallas import tpu_sc as plsc

NC = 2
NS = 16
LB = 128
KB = 8
F32 = jnp.float32


def _round_up(a, b):
    return (a + b - 1) // b * b


def _make_sc_pass(nc, NP, RW, NBAT, gather):
    rows_per_tile = NP // NS
    nzero = rows_per_tile // LB

    mesh = plsc.VectorSubcoreMesh(
        core_axis_name="c", subcore_axis_name="s", num_cores=NC,
        num_subcores=NS)

    out_type = [jax.ShapeDtypeStruct((NC, NP, 16), F32) for _ in range(nc)]
    scratch = [
        pltpu.VMEM((KB, LB), jnp.int32),
        pltpu.VMEM((LB, 16), F32),
        pltpu.VMEM_SHARED((NP, 16), F32),
    ]
    if gather:
        scratch = [pltpu.VMEM((KB, LB), jnp.int32)] + scratch
        scratch.append(pltpu.VMEM((KB, LB, 16), F32))
        scratch.append(pltpu.SemaphoreType.DMA)
        scratch.append(pltpu.SemaphoreType.DMA)
    else:
        scratch.append(pltpu.VMEM((LB, 16), F32))

    def body(*refs):
        if gather:
            s2, d2 = refs[0], refs[1]
            ytabs = refs[2:2 + nc]
            outs = refs[2 + nc:2 + 2 * nc]
            s_buf, d_buf, zbuf, acc, rows, gsem, ssem = refs[2 + 2 * nc:]
        else:
            d2, ones_hbm = refs[0], refs[1]
            outs = refs[2:2 + nc]
            d_buf, zbuf, acc, obuf = refs[2 + nc:]

        cid = lax.axis_index("c")
        sid = lax.axis_index("s")
        my0 = sid * rows_per_tile

        zvec = jnp.zeros((16,), F32)
        for r in range(LB):
            zbuf[r] = zvec
        if not gather:
            pltpu.sync_copy(ones_hbm, obuf)

        def zero_my_slice():
            for k in range(nzero):
                pltpu.sync_copy(zbuf, acc.at[pl.ds(my0 + k * LB, LB)])

        zero_my_slice()
        plsc.subcore_barrier()

        wrow = (cid * NS + sid) * RW

        for chunk in range(nc):
            def batch_body(i, carry):
                r0 = pl.multiple_of(wrow + i * KB, 8)
                pltpu.sync_copy(d2.at[pl.ds(r0, KB)], d_buf)
                if gather:
                    pltpu.sync_copy(s2.at[pl.ds(r0, KB)], s_buf)
                    cps = [
                        pltpu.async_copy(
                            ytabs[chunk].at[s_buf.at[j]], rows.at[j], gsem)
                        for j in range(KB)
                    ]
                    for cp in cps:
                        cp.wait()
                    scps = [
                        pltpu.async_copy(rows.at[j], acc.at[d_buf.at[j]],
                                         ssem, add=True)
                        for j in range(KB)
                    ]
                    for cp in scps:
                        cp.wait()
                else:
                    for j in range(KB):
                        pltpu.sync_copy(obuf, acc.at[d_buf.at[j]], add=True)
                return carry

            lax.fori_loop(0, NBAT, batch_body, 0)
            plsc.subcore_barrier()
            pltpu.sync_copy(acc.at[pl.ds(my0, rows_per_tile)],
                            outs[chunk].at[cid, pl.ds(my0, rows_per_tile)])
            if chunk < nc - 1:
                zero_my_slice()
            plsc.subcore_barrier()

    return pl.kernel(
        body, out_type=out_type, mesh=mesh,
        compiler_params=pltpu.CompilerParams(use_tc_tiling_on_sc=False),
        scratch_types=scratch)


_DOT = dict(preferred_element_type=F32, precision=lax.Precision.HIGHEST)


def _tc_prep(NP, NB, cnt, x_pad, W1p):
    grid = NP // NB

    def body(cnt_ref, x_ref, w_ref, dinv_ref, y1_ref):
        c = cnt_ref[0, :, 0:1] + cnt_ref[1, :, 0:1] + 1.0
        dinv = lax.rsqrt(c)
        dinv16 = jnp.broadcast_to(dinv, (NB, 16))
        dinv_ref[...] = dinv16
        y1_ref[...] = jnp.dot(x_ref[...], w_ref[...], **_DOT) * dinv16

    return pl.pallas_call(
        body,
        grid=(grid,),
        in_specs=[
            pl.BlockSpec((NC, NB, 16), lambda i: (0, i, 0)),
            pl.BlockSpec((NB, 8), lambda i: (i, 0)),
            pl.BlockSpec((8, 16), lambda i: (0, 0)),
        ],
        out_specs=[
            pl.BlockSpec((NB, 16), lambda i: (i, 0)),
            pl.BlockSpec((NB, 16), lambda i: (i, 0)),
        ],
        out_shape=[
            jax.ShapeDtypeStruct((NP, 16), F32),
            jax.ShapeDtypeStruct((NP, 16), F32),
        ],
    )(cnt, x_pad, W1p)


def _tc_mid(NP, NB, nci, nco, accs, ys, dinv16, bt, W):
    grid = NP // NB

    def body(*refs):
        a_refs = refs[:nci]
        y_refs = refs[nci:2 * nci]
        dinv_ref, b_ref, w_ref = refs[2 * nci:2 * nci + 3]
        o_refs = refs[2 * nci + 3:]
        dinv = dinv_ref[...]
        hs = []
        for c in range(nci):
            a = a_refs[c][0] + a_refs[c][1] + y_refs[c][...]
            hs.append(jax.nn.relu(dinv * a + b_ref[0, 16 * c:16 * (c + 1)]))
        for oc in range(nco):
            acc = None
            for c in range(nci):
                blk = w_ref[16 * c:16 * (c + 1), 16 * oc:16 * (oc + 1)]
                t = jnp.dot(hs[c], blk, **_DOT)
                acc = t if acc is None else acc + t
            o_refs[oc][...] = acc * dinv

    in_specs = (
        [pl.BlockSpec((NC, NB, 16), lambda i: (0, i, 0))] * nci
        + [pl.BlockSpec((NB, 16), lambda i: (i, 0))] * nci
        + [pl.BlockSpec((NB, 16), lambda i: (i, 0)),
           pl.BlockSpec((1, 16 * nci), lambda i: (0, 0)),
           pl.BlockSpec((16 * nci, 16 * nco), lambda i: (0, 0))]
    )
    return pl.pallas_call(
        body,
        grid=(grid,),
        in_specs=in_specs,
        out_specs=[pl.BlockSpec((NB, 16), lambda i: (i, 0))] * nco,
        out_shape=[jax.ShapeDtypeStruct((NP, 16), F32)] * nco,
    )(*accs, *ys, dinv16, bt, W)


def _tc_final(NP, NB, G, accs, ys, dinv16, b3t, batch2, Wfc, bfc2):
    grid = NP // NB
    nci = 4

    def body(*refs):
        a_refs = refs[:nci]
        y_refs = refs[nci:2 * nci]
        dinv_ref, b_ref, bat_ref, wfc_ref, bfc_ref = refs[2 * nci:2 * nci + 5]
        out_ref = refs[2 * nci + 5]
        pooled = refs[2 * nci + 6:2 * nci + 6 + nci]
        cnt_ref = refs[2 * nci + 6 + nci]

        i = pl.program_id(0)

        @pl.when(i == 0)
        def _init():
            for c in range(nci):
                pooled[c][...] = jnp.zeros((G, 16), F32)
            cnt_ref[...] = jnp.zeros((G, 1), F32)

        dinv = dinv_ref[...]
        bat = bat_ref[...]
        gid = lax.broadcasted_iota(jnp.int32, (NB, G), 1)
        A = (bat == gid).astype(F32)
        dn = (((0,), (0,)), ((), ()))
        for c in range(nci):
            a = a_refs[c][0] + a_refs[c][1] + y_refs[c][...]
            h = jax.nn.relu(dinv * a + b_ref[0, 16 * c:16 * (c + 1)])
            pooled[c][...] += lax.dot_general(A, h, dn, **_DOT)
        cnt_ref[...] += lax.dot_general(A, jnp.ones((NB, 1), F32), dn, **_DOT)

        @pl.when(i == grid - 1)
        def _fin():
            cnt = jnp.clip(cnt_ref[...], 1.0, None)
            logits = None
            for c in range(nci):
                p = pooled[c][...] / cnt
                t = jnp.dot(p, wfc_ref[16 * c:16 * (c + 1), :], **_DOT)
                logits = t if logits is None else logits + t
            logits = logits + bfc_ref[...]
            m = jnp.max(logits, axis=1, keepdims=True)
            e = jnp.exp(logits - m)
            lse = m + jnp.log(jnp.sum(e, axis=1, keepdims=True))
            out_ref[...] = logits - lse

    in_specs = (
        [pl.BlockSpec((NC, NB, 16), lambda i: (0, i, 0))] * nci
        + [pl.BlockSpec((NB, 16), lambda i: (i, 0))] * nci
        + [pl.BlockSpec((NB, 16), lambda i: (i, 0)),
           pl.BlockSpec((1, 64), lambda i: (0, 0)),
           pl.BlockSpec((NB, 1), lambda i: (i, 0)),
           pl.BlockSpec((64, 2), lambda i: (0, 0)),
           pl.BlockSpec((1, 2), lambda i: (0, 0))]
    )
    return pl.pallas_call(
        body,
        grid=(grid,),
        in_specs=in_specs,
        out_specs=pl.BlockSpec((G, 2), lambda i: (0, 0)),
        out_shape=jax.ShapeDtypeStruct((G, 2), F32),
        scratch_shapes=[pltpu.VMEM((G, 16), F32) for _ in range(nci)]
        + [pltpu.VMEM((G, 1), F32)],
    )(*accs, *ys, dinv16, b3t, batch2, Wfc, bfc2)


def kernel(x, edge_index, batch, W1, b1, W2, b2, W3, b3, Wfc, bfc):
    N = x.shape[0]
    E = edge_index.shape[1]
    G = 128
    NB = 2048
    NP = _round_up(N + 1, NB)
    NW = NC * NS
    per_w = -(-E // NW)
    NBAT = -(-per_w // (KB * LB))
    while (NBAT * KB) % 8:
        NBAT += 1
    RW = NBAT * KB
    EP = NW * RW * LB

    src = edge_index[0]
    dst = edge_index[1]
    pad_e = EP - E
    src2 = jnp.concatenate(
        [src, jnp.full((pad_e,), N, jnp.int32)]).reshape(EP // LB, LB)
    dst2 = jnp.concatenate(
        [dst, jnp.full((pad_e,), N, jnp.int32)]).reshape(EP // LB, LB)

    x_pad = jnp.zeros((NP, 8), F32).at[:N, :5].set(x)
    W1p = jnp.zeros((8, 16), F32).at[:5, :].set(W1)
    batch2 = jnp.concatenate(
        [batch.astype(jnp.int32),
         jnp.full((NP - N,), G, jnp.int32)]).reshape(NP, 1)
    b1t = b1.reshape(1, 16)
    b2t = b2.reshape(1, 32)
    b3t = b3.reshape(1, 64)
    bfc2 = bfc.reshape(1, 2)

    sc_deg = _make_sc_pass(1, NP, RW, NBAT, gather=False)
    sc_l1 = _make_sc_pass(1, NP, RW, NBAT, gather=True)
    sc_l2 = _make_sc_pass(2, NP, RW, NBAT, gather=True)
    sc_l3 = _make_sc_pass(4, NP, RW, NBAT, gather=True)

    ones_rows = jnp.zeros((LB, 16), F32).at[:, 0].set(1.0)
    (dcnt,) = sc_deg(dst2, ones_rows)
    dinv16, y1 = _tc_prep(NP, NB, dcnt, x_pad, W1p)
    (a1,) = sc_l1(src2, dst2, y1)
    y2 = _tc_mid(NP, NB, 1, 2, [a1], [y1], dinv16, b1t, W2)
    a2 = sc_l2(src2, dst2, *y2)
    y3 = _tc_mid(NP, NB, 2, 4, a2, y2, dinv16, b2t, W3)
    a3 = sc_l3(src2, dst2, *y3)
    return _tc_final(NP, NB, G, a3, y3, dinv16, b3t, batch2, Wfc, bfc2)


# speedup vs baseline: 1.5639x; 1.0582x over previous
import jax
import jax.numpy as jnp
from jax import lax
from jax.experimental import pallas as pl
from jax.experimental.pallas import tpu as pltpu
from jax.experimental.pallas import tpu_sc as plsc

NC = 2
NS = 16
LB = 128
KB = 8
F32 = jnp.float32


def _round_up(a, b):
    return (a + b - 1) // b * b


def _make_sc_pass(nc, NP, RW, NBAT, gather):
    rows_per_tile = NP // NS
    nzero = rows_per_tile // LB

    mesh = plsc.VectorSubcoreMesh(
        core_axis_name="c", subcore_axis_name="s", num_cores=NC,
        num_subcores=NS)

    out_type = [jax.ShapeDtypeStruct((NC, NP, 16), F32) for _ in range(nc)]
    scratch = [
        pltpu.VMEM((KB, LB), jnp.int32),
        pltpu.VMEM((LB, 16), F32),
        pltpu.VMEM_SHARED((NP, 16), F32),
        pltpu.SemaphoreType.DMA,
    ]
    if gather:
        HB = KB // 2
        scratch = [pltpu.VMEM((KB, LB), jnp.int32)] + scratch
        scratch.append(pltpu.VMEM((HB, LB, 16), F32))
        scratch.append(pltpu.VMEM((HB, LB, 16), F32))
        scratch.append(pltpu.SemaphoreType.DMA)
        scratch.append(pltpu.SemaphoreType.DMA)
    else:
        scratch.append(pltpu.VMEM((LB, 16), F32))

    def body(*refs):
        if gather:
            s2, d2 = refs[0], refs[1]
            ytabs = refs[2:2 + nc]
            outs = refs[2 + nc:2 + 2 * nc]
            (s_buf, d_buf, zbuf, acc, ssem, rowsA, rowsB, gsemA,
             gsemB) = refs[2 + 2 * nc:]
        else:
            d2, ones_hbm = refs[0], refs[1]
            outs = refs[2:2 + nc]
            d_buf, zbuf, acc, ssem, obuf = refs[2 + nc:]

        cid = lax.axis_index("c")
        sid = lax.axis_index("s")
        my0 = sid * rows_per_tile

        zvec = jnp.zeros((16,), F32)
        for r in range(LB):
            zbuf[r] = zvec
        if not gather:
            pltpu.sync_copy(ones_hbm, obuf)

        def zero_my_slice():
            for k in range(nzero):
                pltpu.sync_copy(zbuf, acc.at[pl.ds(my0 + k * LB, LB)])

        zero_my_slice()
        plsc.subcore_barrier()

        wrow = (cid * NS + sid) * RW

        for chunk in range(nc):
            def batch_body(i, carry):
                r0 = pl.multiple_of(wrow + i * KB, 8)
                pltpu.sync_copy(d2.at[pl.ds(r0, KB)], d_buf)
                if gather:
                    ytab = ytabs[chunk]
                    pltpu.sync_copy(s2.at[pl.ds(r0, KB)], s_buf)
                    gA = [
                        pltpu.async_copy(
                            ytab.at[s_buf.at[j]], rowsA.at[j], gsemA)
                        for j in range(HB)
                    ]
                    gB = [
                        pltpu.async_copy(
                            ytab.at[s_buf.at[HB + j]], rowsB.at[j], gsemB)
                        for j in range(HB)
                    ]
                    for cp in gA:
                        cp.wait()
                    sA = [
                        pltpu.async_copy(rowsA.at[j], acc.at[d_buf.at[j]],
                                         ssem, add=True)
                        for j in range(HB)
                    ]
                    for cp in gB:
                        cp.wait()
                    sB = [
                        pltpu.async_copy(rowsB.at[j],
                                         acc.at[d_buf.at[HB + j]],
                                         ssem, add=True)
                        for j in range(HB)
                    ]
                    for cp in sA:
                        cp.wait()
                    for cp in sB:
                        cp.wait()
                else:
                    scps = [
                        pltpu.async_copy(obuf, acc.at[d_buf.at[j]], ssem,
                                         add=True)
                        for j in range(KB)
                    ]
                    for cp in scps:
                        cp.wait()
                return carry

            lax.fori_loop(0, NBAT, batch_body, 0)
            plsc.subcore_barrier()
            pltpu.sync_copy(acc.at[pl.ds(my0, rows_per_tile)],
                            outs[chunk].at[cid, pl.ds(my0, rows_per_tile)])
            if chunk < nc - 1:
                zero_my_slice()
            plsc.subcore_barrier()

    return pl.kernel(
        body, out_type=out_type, mesh=mesh,
        compiler_params=pltpu.CompilerParams(use_tc_tiling_on_sc=False),
        scratch_types=scratch)


_DOT = dict(preferred_element_type=F32, precision=lax.Precision.HIGHEST)


def _tc_prep(NP, NB, cnt, x_pad, W1p):
    grid = NP // NB

    def body(cnt_ref, x_ref, w_ref, dinv_ref, y1_ref):
        c = cnt_ref[0, :, 0:1] + cnt_ref[1, :, 0:1] + 1.0
        dinv = lax.rsqrt(c)
        dinv16 = jnp.broadcast_to(dinv, (NB, 16))
        dinv_ref[...] = dinv16
        y1_ref[...] = jnp.dot(x_ref[...], w_ref[...], **_DOT) * dinv16

    return pl.pallas_call(
        body,
        grid=(grid,),
        in_specs=[
            pl.BlockSpec((NC, NB, 16), lambda i: (0, i, 0)),
            pl.BlockSpec((NB, 8), lambda i: (i, 0)),
            pl.BlockSpec((8, 16), lambda i: (0, 0)),
        ],
        out_specs=[
            pl.BlockSpec((NB, 16), lambda i: (i, 0)),
            pl.BlockSpec((NB, 16), lambda i: (i, 0)),
        ],
        out_shape=[
            jax.ShapeDtypeStruct((NP, 16), F32),
            jax.ShapeDtypeStruct((NP, 16), F32),
        ],
    )(cnt, x_pad, W1p)


def _tc_mid(NP, NB, nci, nco, accs, ys, dinv16, bt, W):
    grid = NP // NB

    def body(*refs):
        a_refs = refs[:nci]
        y_refs = refs[nci:2 * nci]
        dinv_ref, b_ref, w_ref = refs[2 * nci:2 * nci + 3]
        o_refs = refs[2 * nci + 3:]
        dinv = dinv_ref[...]
        hs = []
        for c in range(nci):
            a = a_refs[c][0] + a_refs[c][1] + y_refs[c][...]
            hs.append(jax.nn.relu(dinv * a + b_ref[0, 16 * c:16 * (c + 1)]))
        for oc in range(nco):
            acc = None
            for c in range(nci):
                blk = w_ref[16 * c:16 * (c + 1), 16 * oc:16 * (oc + 1)]
                t = jnp.dot(hs[c], blk, **_DOT)
                acc = t if acc is None else acc + t
            o_refs[oc][...] = acc * dinv

    in_specs = (
        [pl.BlockSpec((NC, NB, 16), lambda i: (0, i, 0))] * nci
        + [pl.BlockSpec((NB, 16), lambda i: (i, 0))] * nci
        + [pl.BlockSpec((NB, 16), lambda i: (i, 0)),
           pl.BlockSpec((1, 16 * nci), lambda i: (0, 0)),
           pl.BlockSpec((16 * nci, 16 * nco), lambda i: (0, 0))]
    )
    return pl.pallas_call(
        body,
        grid=(grid,),
        in_specs=in_specs,
        out_specs=[pl.BlockSpec((NB, 16), lambda i: (i, 0))] * nco,
        out_shape=[jax.ShapeDtypeStruct((NP, 16), F32)] * nco,
    )(*accs, *ys, dinv16, bt, W)


def _tc_final(NP, NB, G, accs, ys, dinv16, b3t, batch2, Wfc, bfc2):
    grid = NP // NB
    nci = 4

    def body(*refs):
        a_refs = refs[:nci]
        y_refs = refs[nci:2 * nci]
        dinv_ref, b_ref, bat_ref, wfc_ref, bfc_ref = refs[2 * nci:2 * nci + 5]
        out_ref = refs[2 * nci + 5]
        pooled = refs[2 * nci + 6:2 * nci + 6 + nci]
        cnt_ref = refs[2 * nci + 6 + nci]

        i = pl.program_id(0)

        @pl.when(i == 0)
        def _init():
            for c in range(nci):
                pooled[c][...] = jnp.zeros((G, 16), F32)
            cnt_ref[...] = jnp.zeros((G, 1), F32)

        dinv = dinv_ref[...]
        bat = bat_ref[...]
        gid = lax.broadcasted_iota(jnp.int32, (NB, G), 1)
        A = (bat == gid).astype(F32)
        dn = (((0,), (0,)), ((), ()))
        for c in range(nci):
            a = a_refs[c][0] + a_refs[c][1] + y_refs[c][...]
            h = jax.nn.relu(dinv * a + b_ref[0, 16 * c:16 * (c + 1)])
            pooled[c][...] += lax.dot_general(A, h, dn, **_DOT)
        cnt_ref[...] += lax.dot_general(A, jnp.ones((NB, 1), F32), dn, **_DOT)

        @pl.when(i == grid - 1)
        def _fin():
            cnt = jnp.clip(cnt_ref[...], 1.0, None)
            logits = None
            for c in range(nci):
                p = pooled[c][...] / cnt
                t = jnp.dot(p, wfc_ref[16 * c:16 * (c + 1), :], **_DOT)
                logits = t if logits is None else logits + t
            logits = logits + bfc_ref[...]
            m = jnp.max(logits, axis=1, keepdims=True)
            e = jnp.exp(logits - m)
            lse = m + jnp.log(jnp.sum(e, axis=1, keepdims=True))
            out_ref[...] = logits - lse

    in_specs = (
        [pl.BlockSpec((NC, NB, 16), lambda i: (0, i, 0))] * nci
        + [pl.BlockSpec((NB, 16), lambda i: (i, 0))] * nci
        + [pl.BlockSpec((NB, 16), lambda i: (i, 0)),
           pl.BlockSpec((1, 64), lambda i: (0, 0)),
           pl.BlockSpec((NB, 1), lambda i: (i, 0)),
           pl.BlockSpec((64, 2), lambda i: (0, 0)),
           pl.BlockSpec((1, 2), lambda i: (0, 0))]
    )
    return pl.pallas_call(
        body,
        grid=(grid,),
        in_specs=in_specs,
        out_specs=pl.BlockSpec((G, 2), lambda i: (0, 0)),
        out_shape=jax.ShapeDtypeStruct((G, 2), F32),
        scratch_shapes=[pltpu.VMEM((G, 16), F32) for _ in range(nci)]
        + [pltpu.VMEM((G, 1), F32)],
    )(*accs, *ys, dinv16, b3t, batch2, Wfc, bfc2)


def kernel(x, edge_index, batch, W1, b1, W2, b2, W3, b3, Wfc, bfc):
    N = x.shape[0]
    E = edge_index.shape[1]
    G = 128
    NB = 2048
    NP = _round_up(N + 1, NB)
    NW = NC * NS
    per_w = -(-E // NW)
    NBAT = -(-per_w // (KB * LB))
    while (NBAT * KB) % 8:
        NBAT += 1
    RW = NBAT * KB
    EP = NW * RW * LB

    src = edge_index[0]
    dst = edge_index[1]
    pad_e = EP - E
    src2 = jnp.concatenate(
        [src, jnp.full((pad_e,), N, jnp.int32)]).reshape(EP // LB, LB)
    dst2 = jnp.concatenate(
        [dst, jnp.full((pad_e,), N, jnp.int32)]).reshape(EP // LB, LB)

    x_pad = jnp.zeros((NP, 8), F32).at[:N, :5].set(x)
    W1p = jnp.zeros((8, 16), F32).at[:5, :].set(W1)
    batch2 = jnp.concatenate(
        [batch.astype(jnp.int32),
         jnp.full((NP - N,), G, jnp.int32)]).reshape(NP, 1)
    b1t = b1.reshape(1, 16)
    b2t = b2.reshape(1, 32)
    b3t = b3.reshape(1, 64)
    bfc2 = bfc.reshape(1, 2)

    sc_deg = _make_sc_pass(1, NP, RW, NBAT, gather=False)
    sc_l1 = _make_sc_pass(1, NP, RW, NBAT, gather=True)
    sc_l2 = _make_sc_pass(2, NP, RW, NBAT, gather=True)
    sc_l3 = _make_sc_pass(4, NP, RW, NBAT, gather=True)

    ones_rows = jnp.zeros((LB, 16), F32).at[:, 0].set(1.0)
    (dcnt,) = sc_deg(dst2, ones_rows)
    dinv16, y1 = _tc_prep(NP, NB, dcnt, x_pad, W1p)
    (a1,) = sc_l1(src2, dst2, y1)
    y2 = _tc_mid(NP, NB, 1, 2, [a1], [y1], dinv16, b1t, W2)
    a2 = sc_l2(src2, dst2, *y2)
    y3 = _tc_mid(NP, NB, 2, 4, a2, y2, dinv16, b2t, W3)
    a3 = sc_l3(src2, dst2, *y3)
    return _tc_final(NP, NB, G, a3, y3, dinv16, b3t, batch2, Wfc, bfc2)


# speedup vs baseline: 1.6026x; 1.0247x over previous
import jax
import jax.numpy as jnp
from jax import lax
from jax.experimental import pallas as pl
from jax.experimental.pallas import tpu as pltpu
from jax.experimental.pallas import tpu_sc as plsc

NC = 2
NS = 16
LB = 128
KB = 8
F32 = jnp.float32


def _round_up(a, b):
    return (a + b - 1) // b * b


def _make_sc_pass(nc, NP, RW, NBAT, gather):
    rows_per_tile = NP // NS
    nzero = rows_per_tile // LB

    mesh = plsc.VectorSubcoreMesh(
        core_axis_name="c", subcore_axis_name="s", num_cores=NC,
        num_subcores=NS)

    out_type = [jax.ShapeDtypeStruct((NC, NP, 16), F32) for _ in range(nc)]
    scratch = [
        pltpu.VMEM((KB, LB), jnp.int32),
        pltpu.VMEM((LB, 16), F32),
        pltpu.VMEM_SHARED((NP, 16), F32),
        pltpu.SemaphoreType.DMA,
    ]
    if gather:
        HB = KB // 2
        scratch = [pltpu.VMEM((KB, LB), jnp.int32)] + scratch
        scratch.append(pltpu.VMEM((HB, LB, 16), F32))
        scratch.append(pltpu.VMEM((HB, LB, 16), F32))
        scratch.append(pltpu.SemaphoreType.DMA)
        scratch.append(pltpu.SemaphoreType.DMA)
    else:
        scratch.append(pltpu.VMEM((LB, 16), F32))

    def body(*refs):
        if gather:
            s2, d2 = refs[0], refs[1]
            ytabs = refs[2:2 + nc]
            outs = refs[2 + nc:2 + 2 * nc]
            (s_buf, d_buf, zbuf, acc, ssem, rowsA, rowsB, gsemA,
             gsemB) = refs[2 + 2 * nc:]
        else:
            d2, ones_hbm = refs[0], refs[1]
            outs = refs[2:2 + nc]
            d_buf, zbuf, acc, ssem, obuf = refs[2 + nc:]

        cid = lax.axis_index("c")
        sid = lax.axis_index("s")
        my0 = sid * rows_per_tile

        zvec = jnp.zeros((16,), F32)
        for r in range(LB):
            zbuf[r] = zvec
        if not gather:
            pltpu.sync_copy(ones_hbm, obuf)

        def zero_my_slice():
            for k in range(nzero):
                pltpu.sync_copy(zbuf, acc.at[pl.ds(my0 + k * LB, LB)])

        zero_my_slice()
        plsc.subcore_barrier()

        wrow = (cid * NS + sid) * RW

        for chunk in range(nc):
            def batch_body(i, carry):
                r0 = pl.multiple_of(wrow + i * KB, 8)
                pltpu.sync_copy(d2.at[pl.ds(r0, KB)], d_buf)
                if gather:
                    ytab = ytabs[chunk]
                    pltpu.sync_copy(s2.at[pl.ds(r0, KB)], s_buf)
                    gA = [
                        pltpu.async_copy(
                            ytab.at[s_buf.at[j]], rowsA.at[j], gsemA)
                        for j in range(HB)
                    ]
                    gB = [
                        pltpu.async_copy(
                            ytab.at[s_buf.at[HB + j]], rowsB.at[j], gsemB)
                        for j in range(HB)
                    ]
                    sA = []
                    for j in range(HB):
                        gA[j].wait()
                        sA.append(
                            pltpu.async_copy(rowsA.at[j],
                                             acc.at[d_buf.at[j]],
                                             ssem, add=True))
                    sB = []
                    for j in range(HB):
                        gB[j].wait()
                        sB.append(
                            pltpu.async_copy(rowsB.at[j],
                                             acc.at[d_buf.at[HB + j]],
                                             ssem, add=True))
                    for cp in sA:
                        cp.wait()
                    for cp in sB:
                        cp.wait()
                else:
                    scps = [
                        pltpu.async_copy(obuf, acc.at[d_buf.at[j]], ssem,
                                         add=True)
                        for j in range(KB)
                    ]
                    for cp in scps:
                        cp.wait()
                return carry

            lax.fori_loop(0, NBAT, batch_body, 0)
            plsc.subcore_barrier()
            pltpu.sync_copy(acc.at[pl.ds(my0, rows_per_tile)],
                            outs[chunk].at[cid, pl.ds(my0, rows_per_tile)])
            if chunk < nc - 1:
                zero_my_slice()
            plsc.subcore_barrier()

    return pl.kernel(
        body, out_type=out_type, mesh=mesh,
        compiler_params=pltpu.CompilerParams(use_tc_tiling_on_sc=False),
        scratch_types=scratch)


_DOT = dict(preferred_element_type=F32, precision=lax.Precision.HIGHEST)


def _tc_prep(NP, NB, cnt, x_pad, W1p):
    grid = NP // NB

    def body(cnt_ref, x_ref, w_ref, dinv_ref, y1_ref):
        c = cnt_ref[0, :, 0:1] + cnt_ref[1, :, 0:1] + 1.0
        dinv = lax.rsqrt(c)
        dinv16 = jnp.broadcast_to(dinv, (NB, 16))
        dinv_ref[...] = dinv16
        y1_ref[...] = jnp.dot(x_ref[...], w_ref[...], **_DOT) * dinv16

    return pl.pallas_call(
        body,
        grid=(grid,),
        in_specs=[
            pl.BlockSpec((NC, NB, 16), lambda i: (0, i, 0)),
            pl.BlockSpec((NB, 8), lambda i: (i, 0)),
            pl.BlockSpec((8, 16), lambda i: (0, 0)),
        ],
        out_specs=[
            pl.BlockSpec((NB, 16), lambda i: (i, 0)),
            pl.BlockSpec((NB, 16), lambda i: (i, 0)),
        ],
        out_shape=[
            jax.ShapeDtypeStruct((NP, 16), F32),
            jax.ShapeDtypeStruct((NP, 16), F32),
        ],
    )(cnt, x_pad, W1p)


def _tc_mid(NP, NB, nci, nco, accs, ys, dinv16, bt, W):
    grid = NP // NB

    def body(*refs):
        a_refs = refs[:nci]
        y_refs = refs[nci:2 * nci]
        dinv_ref, b_ref, w_ref = refs[2 * nci:2 * nci + 3]
        o_refs = refs[2 * nci + 3:]
        dinv = dinv_ref[...]
        hs = []
        for c in range(nci):
            a = a_refs[c][0] + a_refs[c][1] + y_refs[c][...]
            hs.append(jax.nn.relu(dinv * a + b_ref[0, 16 * c:16 * (c + 1)]))
        for oc in range(nco):
            acc = None
            for c in range(nci):
                blk = w_ref[16 * c:16 * (c + 1), 16 * oc:16 * (oc + 1)]
                t = jnp.dot(hs[c], blk, **_DOT)
                acc = t if acc is None else acc + t
            o_refs[oc][...] = acc * dinv

    in_specs = (
        [pl.BlockSpec((NC, NB, 16), lambda i: (0, i, 0))] * nci
        + [pl.BlockSpec((NB, 16), lambda i: (i, 0))] * nci
        + [pl.BlockSpec((NB, 16), lambda i: (i, 0)),
           pl.BlockSpec((1, 16 * nci), lambda i: (0, 0)),
           pl.BlockSpec((16 * nci, 16 * nco), lambda i: (0, 0))]
    )
    return pl.pallas_call(
        body,
        grid=(grid,),
        in_specs=in_specs,
        out_specs=[pl.BlockSpec((NB, 16), lambda i: (i, 0))] * nco,
        out_shape=[jax.ShapeDtypeStruct((NP, 16), F32)] * nco,
    )(*accs, *ys, dinv16, bt, W)


def _tc_final(NP, NB, G, accs, ys, dinv16, b3t, batch2, Wfc, bfc2):
    grid = NP // NB
    nci = 4

    def body(*refs):
        a_refs = refs[:nci]
        y_refs = refs[nci:2 * nci]
        dinv_ref, b_ref, bat_ref, wfc_ref, bfc_ref = refs[2 * nci:2 * nci + 5]
        out_ref = refs[2 * nci + 5]
        pooled = refs[2 * nci + 6:2 * nci + 6 + nci]
        cnt_ref = refs[2 * nci + 6 + nci]

        i = pl.program_id(0)

        @pl.when(i == 0)
        def _init():
            for c in range(nci):
                pooled[c][...] = jnp.zeros((G, 16), F32)
            cnt_ref[...] = jnp.zeros((G, 1), F32)

        dinv = dinv_ref[...]
        bat = bat_ref[...]
        gid = lax.broadcasted_iota(jnp.int32, (NB, G), 1)
        A = (bat == gid).astype(F32)
        dn = (((0,), (0,)), ((), ()))
        for c in range(nci):
            a = a_refs[c][0] + a_refs[c][1] + y_refs[c][...]
            h = jax.nn.relu(dinv * a + b_ref[0, 16 * c:16 * (c + 1)])
            pooled[c][...] += lax.dot_general(A, h, dn, **_DOT)
        cnt_ref[...] += lax.dot_general(A, jnp.ones((NB, 1), F32), dn, **_DOT)

        @pl.when(i == grid - 1)
        def _fin():
            cnt = jnp.clip(cnt_ref[...], 1.0, None)
            logits = None
            for c in range(nci):
                p = pooled[c][...] / cnt
                t = jnp.dot(p, wfc_ref[16 * c:16 * (c + 1), :], **_DOT)
                logits = t if logits is None else logits + t
            logits = logits + bfc_ref[...]
            m = jnp.max(logits, axis=1, keepdims=True)
            e = jnp.exp(logits - m)
            lse = m + jnp.log(jnp.sum(e, axis=1, keepdims=True))
            out_ref[...] = logits - lse

    in_specs = (
        [pl.BlockSpec((NC, NB, 16), lambda i: (0, i, 0))] * nci
        + [pl.BlockSpec((NB, 16), lambda i: (i, 0))] * nci
        + [pl.BlockSpec((NB, 16), lambda i: (i, 0)),
           pl.BlockSpec((1, 64), lambda i: (0, 0)),
           pl.BlockSpec((NB, 1), lambda i: (i, 0)),
           pl.BlockSpec((64, 2), lambda i: (0, 0)),
           pl.BlockSpec((1, 2), lambda i: (0, 0))]
    )
    return pl.pallas_call(
        body,
        grid=(grid,),
        in_specs=in_specs,
        out_specs=pl.BlockSpec((G, 2), lambda i: (0, 0)),
        out_shape=jax.ShapeDtypeStruct((G, 2), F32),
        scratch_shapes=[pltpu.VMEM((G, 16), F32) for _ in range(nci)]
        + [pltpu.VMEM((G, 1), F32)],
    )(*accs, *ys, dinv16, b3t, batch2, Wfc, bfc2)


def kernel(x, edge_index, batch, W1, b1, W2, b2, W3, b3, Wfc, bfc):
    N = x.shape[0]
    E = edge_index.shape[1]
    G = 128
    NB = 2048
    NP = _round_up(N + 1, NB)
    NW = NC * NS
    per_w = -(-E // NW)
    NBAT = -(-per_w // (KB * LB))
    while (NBAT * KB) % 8:
        NBAT += 1
    RW = NBAT * KB
    EP = NW * RW * LB

    src = edge_index[0]
    dst = edge_index[1]
    pad_e = EP - E
    src2 = jnp.concatenate(
        [src, jnp.full((pad_e,), N, jnp.int32)]).reshape(EP // LB, LB)
    dst2 = jnp.concatenate(
        [dst, jnp.full((pad_e,), N, jnp.int32)]).reshape(EP // LB, LB)

    x_pad = jnp.zeros((NP, 8), F32).at[:N, :5].set(x)
    W1p = jnp.zeros((8, 16), F32).at[:5, :].set(W1)
    batch2 = jnp.concatenate(
        [batch.astype(jnp.int32),
         jnp.full((NP - N,), G, jnp.int32)]).reshape(NP, 1)
    b1t = b1.reshape(1, 16)
    b2t = b2.reshape(1, 32)
    b3t = b3.reshape(1, 64)
    bfc2 = bfc.reshape(1, 2)

    sc_deg = _make_sc_pass(1, NP, RW, NBAT, gather=False)
    sc_l1 = _make_sc_pass(1, NP, RW, NBAT, gather=True)
    sc_l2 = _make_sc_pass(2, NP, RW, NBAT, gather=True)
    sc_l3 = _make_sc_pass(4, NP, RW, NBAT, gather=True)

    ones_rows = jnp.zeros((LB, 16), F32).at[:, 0].set(1.0)
    (dcnt,) = sc_deg(dst2, ones_rows)
    dinv16, y1 = _tc_prep(NP, NB, dcnt, x_pad, W1p)
    (a1,) = sc_l1(src2, dst2, y1)
    y2 = _tc_mid(NP, NB, 1, 2, [a1], [y1], dinv16, b1t, W2)
    a2 = sc_l2(src2, dst2, *y2)
    y3 = _tc_mid(NP, NB, 2, 4, a2, y2, dinv16, b2t, W3)
    a3 = sc_l3(src2, dst2, *y3)
    return _tc_final(NP, NB, G, a3, y3, dinv16, b3t, batch2, Wfc, bfc2)


# speedup vs baseline: 1.6764x; 1.0461x over previous
import jax
import jax.numpy as jnp
from jax import lax
from jax.experimental import pallas as pl
from jax.experimental.pallas import tpu as pltpu
from jax.experimental.pallas import tpu_sc as plsc

NC = 2
NS = 16
LB = 128
KB = 8
F32 = jnp.float32


def _round_up(a, b):
    return (a + b - 1) // b * b


def _make_sc_pass(nc, NP, RW, NBAT, gather):
    rows_per_tile = NP // NS
    nzero = rows_per_tile // LB

    mesh = plsc.VectorSubcoreMesh(
        core_axis_name="c", subcore_axis_name="s", num_cores=NC,
        num_subcores=NS)

    out_type = [jax.ShapeDtypeStruct((NC, NP, 16), F32) for _ in range(nc)]
    scratch = [
        pltpu.VMEM((KB, LB), jnp.int32),
        pltpu.VMEM((LB, 16), F32),
        pltpu.VMEM_SHARED((NP, 16), F32),
        pltpu.SemaphoreType.DMA,
    ]
    if gather:
        HB = KB // 2
        scratch = [
            pltpu.VMEM((2 * KB, LB), jnp.int32),
            pltpu.VMEM((LB, 16), F32),
            pltpu.VMEM_SHARED((NP, 16), F32),
            pltpu.SemaphoreType.DMA,
        ]
        scratch.append(pltpu.VMEM((HB, LB, 16), F32))
        scratch.append(pltpu.VMEM((HB, LB, 16), F32))
        scratch.append(pltpu.SemaphoreType.DMA)
        scratch.append(pltpu.SemaphoreType.DMA)
    else:
        scratch.append(pltpu.VMEM((LB, 16), F32))

    def body(*refs):
        if gather:
            sd2 = refs[0]
            ytabs = refs[1:1 + nc]
            outs = refs[1 + nc:1 + 2 * nc]
            (sd_buf, zbuf, acc, ssem, rowsA, rowsB, gsemA,
             gsemB) = refs[1 + 2 * nc:]
        else:
            d2, ones_hbm = refs[0], refs[1]
            outs = refs[2:2 + nc]
            d_buf, zbuf, acc, ssem, obuf = refs[2 + nc:]

        cid = lax.axis_index("c")
        sid = lax.axis_index("s")
        my0 = sid * rows_per_tile

        zvec = jnp.zeros((16,), F32)
        for r in range(LB):
            zbuf[r] = zvec
        if not gather:
            pltpu.sync_copy(ones_hbm, obuf)

        def zero_my_slice():
            for k in range(nzero):
                pltpu.sync_copy(zbuf, acc.at[pl.ds(my0 + k * LB, LB)])

        zero_my_slice()
        plsc.subcore_barrier()

        wrow = (cid * NS + sid) * RW

        for chunk in range(nc):
            def batch_body(i, carry):
                if gather:
                    ytab = ytabs[chunk]
                    r0 = pl.multiple_of((wrow + i * KB) * 2, 16)
                    pltpu.sync_copy(sd2.at[pl.ds(r0, 2 * KB)], sd_buf)
                    gA = [
                        pltpu.async_copy(
                            ytab.at[sd_buf.at[2 * j]], rowsA.at[j], gsemA)
                        for j in range(HB)
                    ]
                    gB = [
                        pltpu.async_copy(
                            ytab.at[sd_buf.at[2 * (HB + j)]], rowsB.at[j],
                            gsemB)
                        for j in range(HB)
                    ]
                    sA = []
                    for j in range(HB):
                        gA[j].wait()
                        sA.append(
                            pltpu.async_copy(rowsA.at[j],
                                             acc.at[sd_buf.at[2 * j + 1]],
                                             ssem, add=True))
                    sB = []
                    for j in range(HB):
                        gB[j].wait()
                        sB.append(
                            pltpu.async_copy(
                                rowsB.at[j],
                                acc.at[sd_buf.at[2 * (HB + j) + 1]],
                                ssem, add=True))
                    for cp in sA:
                        cp.wait()
                    for cp in sB:
                        cp.wait()
                else:
                    r0 = pl.multiple_of(wrow + i * KB, 8)
                    pltpu.sync_copy(d2.at[pl.ds(r0, KB)], d_buf)
                    scps = [
                        pltpu.async_copy(obuf, acc.at[d_buf.at[j]], ssem,
                                         add=True)
                        for j in range(KB)
                    ]
                    for cp in scps:
                        cp.wait()
                return carry

            lax.fori_loop(0, NBAT, batch_body, 0)
            plsc.subcore_barrier()
            pltpu.sync_copy(acc.at[pl.ds(my0, rows_per_tile)],
                            outs[chunk].at[cid, pl.ds(my0, rows_per_tile)])
            if chunk < nc - 1:
                zero_my_slice()
            plsc.subcore_barrier()

    return pl.kernel(
        body, out_type=out_type, mesh=mesh,
        compiler_params=pltpu.CompilerParams(use_tc_tiling_on_sc=False),
        scratch_types=scratch)


_DOT = dict(preferred_element_type=F32, precision=lax.Precision.HIGHEST)


def _tc_prep(NP, NB, cnt, x_pad, W1p):
    grid = NP // NB

    def body(cnt_ref, x_ref, w_ref, dinv_ref, y1_ref):
        c = cnt_ref[0, :, 0:1] + cnt_ref[1, :, 0:1] + 1.0
        dinv = lax.rsqrt(c)
        dinv16 = jnp.broadcast_to(dinv, (NB, 16))
        dinv_ref[...] = dinv16
        y1_ref[...] = jnp.dot(x_ref[...], w_ref[...], **_DOT) * dinv16

    return pl.pallas_call(
        body,
        grid=(grid,),
        in_specs=[
            pl.BlockSpec((NC, NB, 16), lambda i: (0, i, 0)),
            pl.BlockSpec((NB, 8), lambda i: (i, 0)),
            pl.BlockSpec((8, 16), lambda i: (0, 0)),
        ],
        out_specs=[
            pl.BlockSpec((NB, 16), lambda i: (i, 0)),
            pl.BlockSpec((NB, 16), lambda i: (i, 0)),
        ],
        out_shape=[
            jax.ShapeDtypeStruct((NP, 16), F32),
            jax.ShapeDtypeStruct((NP, 16), F32),
        ],
    )(cnt, x_pad, W1p)


def _tc_mid(NP, NB, nci, nco, accs, ys, dinv16, bt, W):
    grid = NP // NB

    def body(*refs):
        a_refs = refs[:nci]
        y_refs = refs[nci:2 * nci]
        dinv_ref, b_ref, w_ref = refs[2 * nci:2 * nci + 3]
        o_refs = refs[2 * nci + 3:]
        dinv = dinv_ref[...]
        hs = []
        for c in range(nci):
            a = a_refs[c][0] + a_refs[c][1] + y_refs[c][...]
            hs.append(jax.nn.relu(dinv * a + b_ref[0, 16 * c:16 * (c + 1)]))
        for oc in range(nco):
            acc = None
            for c in range(nci):
                blk = w_ref[16 * c:16 * (c + 1), 16 * oc:16 * (oc + 1)]
                t = jnp.dot(hs[c], blk, **_DOT)
                acc = t if acc is None else acc + t
            o_refs[oc][...] = acc * dinv

    in_specs = (
        [pl.BlockSpec((NC, NB, 16), lambda i: (0, i, 0))] * nci
        + [pl.BlockSpec((NB, 16), lambda i: (i, 0))] * nci
        + [pl.BlockSpec((NB, 16), lambda i: (i, 0)),
           pl.BlockSpec((1, 16 * nci), lambda i: (0, 0)),
           pl.BlockSpec((16 * nci, 16 * nco), lambda i: (0, 0))]
    )
    return pl.pallas_call(
        body,
        grid=(grid,),
        in_specs=in_specs,
        out_specs=[pl.BlockSpec((NB, 16), lambda i: (i, 0))] * nco,
        out_shape=[jax.ShapeDtypeStruct((NP, 16), F32)] * nco,
    )(*accs, *ys, dinv16, bt, W)


def _tc_final(NP, NB, G, accs, ys, dinv16, b3t, batch2, Wfc, bfc2):
    grid = NP // NB
    nci = 4

    def body(*refs):
        a_refs = refs[:nci]
        y_refs = refs[nci:2 * nci]
        dinv_ref, b_ref, bat_ref, wfc_ref, bfc_ref = refs[2 * nci:2 * nci + 5]
        out_ref = refs[2 * nci + 5]
        pooled = refs[2 * nci + 6:2 * nci + 6 + nci]
        cnt_ref = refs[2 * nci + 6 + nci]

        i = pl.program_id(0)

        @pl.when(i == 0)
        def _init():
            for c in range(nci):
                pooled[c][...] = jnp.zeros((G, 16), F32)
            cnt_ref[...] = jnp.zeros((G, 1), F32)

        dinv = dinv_ref[...]
        bat = bat_ref[...]
        gid = lax.broadcasted_iota(jnp.int32, (NB, G), 1)
        A = (bat == gid).astype(F32)
        dn = (((0,), (0,)), ((), ()))
        for c in range(nci):
            a = a_refs[c][0] + a_refs[c][1] + y_refs[c][...]
            h = jax.nn.relu(dinv * a + b_ref[0, 16 * c:16 * (c + 1)])
            pooled[c][...] += lax.dot_general(A, h, dn, **_DOT)
        cnt_ref[...] += lax.dot_general(A, jnp.ones((NB, 1), F32), dn, **_DOT)

        @pl.when(i == grid - 1)
        def _fin():
            cnt = jnp.clip(cnt_ref[...], 1.0, None)
            logits = None
            for c in range(nci):
                p = pooled[c][...] / cnt
                t = jnp.dot(p, wfc_ref[16 * c:16 * (c + 1), :], **_DOT)
                logits = t if logits is None else logits + t
            logits = logits + bfc_ref[...]
            m = jnp.max(logits, axis=1, keepdims=True)
            e = jnp.exp(logits - m)
            lse = m + jnp.log(jnp.sum(e, axis=1, keepdims=True))
            out_ref[...] = logits - lse

    in_specs = (
        [pl.BlockSpec((NC, NB, 16), lambda i: (0, i, 0))] * nci
        + [pl.BlockSpec((NB, 16), lambda i: (i, 0))] * nci
        + [pl.BlockSpec((NB, 16), lambda i: (i, 0)),
           pl.BlockSpec((1, 64), lambda i: (0, 0)),
           pl.BlockSpec((NB, 1), lambda i: (i, 0)),
           pl.BlockSpec((64, 2), lambda i: (0, 0)),
           pl.BlockSpec((1, 2), lambda i: (0, 0))]
    )
    return pl.pallas_call(
        body,
        grid=(grid,),
        in_specs=in_specs,
        out_specs=pl.BlockSpec((G, 2), lambda i: (0, 0)),
        out_shape=jax.ShapeDtypeStruct((G, 2), F32),
        scratch_shapes=[pltpu.VMEM((G, 16), F32) for _ in range(nci)]
        + [pltpu.VMEM((G, 1), F32)],
    )(*accs, *ys, dinv16, b3t, batch2, Wfc, bfc2)


def kernel(x, edge_index, batch, W1, b1, W2, b2, W3, b3, Wfc, bfc):
    N = x.shape[0]
    E = edge_index.shape[1]
    G = 128
    NB = 2048
    NP = _round_up(N + 1, NB)
    NW = NC * NS
    per_w = -(-E // NW)
    NBAT = -(-per_w // (KB * LB))
    while (NBAT * KB) % 8:
        NBAT += 1
    RW = NBAT * KB
    EP = NW * RW * LB

    src = edge_index[0]
    dst = edge_index[1]
    pad_e = EP - E
    src2 = jnp.concatenate(
        [src, jnp.full((pad_e,), N, jnp.int32)]).reshape(EP // LB, LB)
    dst2 = jnp.concatenate(
        [dst, jnp.full((pad_e,), N, jnp.int32)]).reshape(EP // LB, LB)
    sd2 = jnp.stack([src2, dst2], axis=1).reshape(2 * (EP // LB), LB)

    x_pad = jnp.zeros((NP, 8), F32).at[:N, :5].set(x)
    W1p = jnp.zeros((8, 16), F32).at[:5, :].set(W1)
    batch2 = jnp.concatenate(
        [batch.astype(jnp.int32),
         jnp.full((NP - N,), G, jnp.int32)]).reshape(NP, 1)
    b1t = b1.reshape(1, 16)
    b2t = b2.reshape(1, 32)
    b3t = b3.reshape(1, 64)
    bfc2 = bfc.reshape(1, 2)

    sc_deg = _make_sc_pass(1, NP, RW, NBAT, gather=False)
    sc_l1 = _make_sc_pass(1, NP, RW, NBAT, gather=True)
    sc_l2 = _make_sc_pass(2, NP, RW, NBAT, gather=True)
    sc_l3 = _make_sc_pass(4, NP, RW, NBAT, gather=True)

    ones_rows = jnp.zeros((LB, 16), F32).at[:, 0].set(1.0)
    (dcnt,) = sc_deg(dst2, ones_rows)
    dinv16, y1 = _tc_prep(NP, NB, dcnt, x_pad, W1p)
    (a1,) = sc_l1(sd2, y1)
    y2 = _tc_mid(NP, NB, 1, 2, [a1], [y1], dinv16, b1t, W2)
    a2 = sc_l2(sd2, *y2)
    y3 = _tc_mid(NP, NB, 2, 4, a2, y2, dinv16, b2t, W3)
    a3 = sc_l3(sd2, *y3)
    return _tc_final(NP, NB, G, a3, y3, dinv16, b3t, batch2, Wfc, bfc2)
